# baseline (device time: 17084 ns/iter reference)
import jax
import jax.numpy as jnp
from jax import lax
from jax.experimental import pallas as pl
from jax.experimental.pallas import tpu as pltpu

NCHUNK = 4


def kernel(x, dy, gamma):
    m, d = x.shape
    cm = m // NCHUNK

    def body(x_hbm, dy_hbm, out_ref, xbuf, dybuf, acc_ref, recv_ref,
             copy_sems, send_sem, recv_sem):
        copies = []
        for c in range(NCHUNK):
            cx = pltpu.make_async_copy(
                x_hbm.at[pl.ds(c * cm, cm), :], xbuf.at[c], copy_sems.at[0, c]
            )
            cd = pltpu.make_async_copy(
                dy_hbm.at[pl.ds(c * cm, cm), :], dybuf.at[c], copy_sems.at[1, c]
            )
            cx.start()
            cd.start()
            copies.append((cx, cd))

        for c in range(NCHUNK):
            cx, cd = copies[c]
            cx.wait()
            cd.wait()
            xv = xbuf[c]
            dyv = dybuf[c]
            b = jnp.sum(dyv, axis=0, keepdims=True)
            g = b + jnp.sum(xv, axis=0, keepdims=True)
            if c == 0:
                acc_ref[0:1, :] = g
                acc_ref[1:2, :] = b
            else:
                acc_ref[0:1, :] += g
                acc_ref[1:2, :] += b

        my_x = lax.axis_index("x")
        my_y = lax.axis_index("y")
        peer = (1 - my_x, my_y)

        barrier_sem = pltpu.get_barrier_semaphore()
        pl.semaphore_signal(
            barrier_sem, inc=1, device_id=peer,
            device_id_type=pl.DeviceIdType.MESH,
        )
        pl.semaphore_wait(barrier_sem, 1)

        rdma = pltpu.make_async_remote_copy(
            src_ref=acc_ref,
            dst_ref=recv_ref,
            send_sem=send_sem,
            recv_sem=recv_sem,
            device_id=peer,
            device_id_type=pl.DeviceIdType.MESH,
        )
        rdma.start()
        rdma.wait()

        out_ref[:, :] = acc_ref[:, :] + recv_ref[:, :]

    return pl.pallas_call(
        body,
        out_shape=jax.ShapeDtypeStruct((2, d), jnp.float32),
        in_specs=[
            pl.BlockSpec(memory_space=pltpu.MemorySpace.HBM),
            pl.BlockSpec(memory_space=pltpu.MemorySpace.HBM),
        ],
        out_specs=pl.BlockSpec(memory_space=pltpu.VMEM),
        scratch_shapes=[
            pltpu.VMEM((NCHUNK, cm, d), jnp.float32),
            pltpu.VMEM((NCHUNK, cm, d), jnp.float32),
            pltpu.VMEM((2, d), jnp.float32),
            pltpu.VMEM((2, d), jnp.float32),
            pltpu.SemaphoreType.DMA((2, NCHUNK)),
            pltpu.SemaphoreType.DMA,
            pltpu.SemaphoreType.DMA,
        ],
        compiler_params=pltpu.CompilerParams(
            collective_id=0, vmem_limit_bytes=60 * 1024 * 1024
        ),
    )(x, dy)


# device time: 13891 ns/iter; 1.2299x vs baseline; 1.2299x over previous
import jax
import jax.numpy as jnp
from jax import lax
from jax.experimental import pallas as pl
from jax.experimental.pallas import tpu as pltpu

NCHUNK = 4


def kernel(x, dy, gamma):
    m, d = x.shape
    half = m // 2
    cm = half // NCHUNK

    def body(x_hbm, dy_hbm, out_ref, xbuf, dybuf, acc_ref, recv_ref,
             copy_sems, send_sems, recv_sems):
        my_x = lax.axis_index("x")
        my_y = lax.axis_index("y")
        base = my_y * half
        peers = [(1 - my_x, my_y), (my_x, 1 - my_y), (1 - my_x, 1 - my_y)]

        copies = []
        for c in range(NCHUNK):
            cx = pltpu.make_async_copy(
                x_hbm.at[pl.ds(base + c * cm, cm), :], xbuf.at[c],
                copy_sems.at[0, c],
            )
            cd = pltpu.make_async_copy(
                dy_hbm.at[pl.ds(base + c * cm, cm), :], dybuf.at[c],
                copy_sems.at[1, c],
            )
            cx.start()
            cd.start()
            copies.append((cx, cd))

        barrier_sem = pltpu.get_barrier_semaphore()
        for p in peers:
            pl.semaphore_signal(
                barrier_sem, inc=1, device_id=p,
                device_id_type=pl.DeviceIdType.MESH,
            )
        pl.semaphore_wait(barrier_sem, 3)

        for c in range(NCHUNK):
            cx, cd = copies[c]
            cx.wait()
            cd.wait()
            xv = xbuf[c]
            dyv = dybuf[c]
            mu = jnp.mean(xv, axis=1, keepdims=True)
            var = jnp.mean(xv * xv, axis=1, keepdims=True) - mu * mu
            rstd = lax.rsqrt(var + 1e-5)
            g = jnp.sum(dyv * ((xv - mu) * rstd), axis=0, keepdims=True)
            b = jnp.sum(dyv, axis=0, keepdims=True)
            if c == 0:
                acc_ref[0:1, :] = g
                acc_ref[1:2, :] = b
            else:
                acc_ref[0:1, :] += g
                acc_ref[1:2, :] += b

        rdmas = []
        for k, p in enumerate(peers):
            r = pltpu.make_async_remote_copy(
                src_ref=acc_ref,
                dst_ref=recv_ref.at[k],
                send_sem=send_sems.at[k],
                recv_sem=recv_sems.at[k],
                device_id=p,
                device_id_type=pl.DeviceIdType.MESH,
            )
            r.start()
            rdmas.append(r)
        for r in rdmas:
            r.wait()

        out_ref[:, :] = (
            (acc_ref[:, :] + recv_ref[0]) + (recv_ref[1] + recv_ref[2])
        )

    return pl.pallas_call(
        body,
        out_shape=jax.ShapeDtypeStruct((2, d), jnp.float32),
        in_specs=[
            pl.BlockSpec(memory_space=pltpu.MemorySpace.HBM),
            pl.BlockSpec(memory_space=pltpu.MemorySpace.HBM),
        ],
        out_specs=pl.BlockSpec(memory_space=pltpu.VMEM),
        scratch_shapes=[
            pltpu.VMEM((NCHUNK, cm, d), jnp.float32),
            pltpu.VMEM((NCHUNK, cm, d), jnp.float32),
            pltpu.VMEM((2, d), jnp.float32),
            pltpu.VMEM((3, 2, d), jnp.float32),
            pltpu.SemaphoreType.DMA((2, NCHUNK)),
            pltpu.SemaphoreType.DMA((3,)),
            pltpu.SemaphoreType.DMA((3,)),
        ],
        compiler_params=pltpu.CompilerParams(
            collective_id=0, vmem_limit_bytes=60 * 1024 * 1024
        ),
    )(x, dy)


# device time: 13876 ns/iter; 1.2312x vs baseline; 1.0011x over previous
import jax
import jax.numpy as jnp
from jax import lax
from jax.experimental import pallas as pl
from jax.experimental.pallas import tpu as pltpu

NCHUNK = 4


def kernel(x, dy, gamma):
    m, d = x.shape
    half = m // 2
    cm = half // NCHUNK

    def body(x_hbm, dy_hbm, out_ref, xbuf, dybuf, acc_ref, recv_ref,
             copy_sems, send_sems, recv_sems):
        my_x = lax.axis_index("x")
        my_y = lax.axis_index("y")
        base = my_y * half
        peers = [(1 - my_x, my_y), (my_x, 1 - my_y), (1 - my_x, 1 - my_y)]

        copies = []
        for c in range(NCHUNK):
            cx = pltpu.make_async_copy(
                x_hbm.at[pl.ds(base + c * cm, cm), :], xbuf.at[c],
                copy_sems.at[0, c],
            )
            cd = pltpu.make_async_copy(
                dy_hbm.at[pl.ds(base + c * cm, cm), :], dybuf.at[c],
                copy_sems.at[1, c],
            )
            cx.start()
            cd.start()
            copies.append((cx, cd))

        barrier_sem = pltpu.get_barrier_semaphore()
        for p in peers:
            pl.semaphore_signal(
                barrier_sem, inc=1, device_id=p,
                device_id_type=pl.DeviceIdType.MESH,
            )

        for c in range(NCHUNK):
            cx, cd = copies[c]
            cx.wait()
            cd.wait()
            xv = xbuf[c]
            dyv = dybuf[c]
            mu = jnp.mean(xv, axis=1, keepdims=True)
            var = jnp.mean(xv * xv, axis=1, keepdims=True) - mu * mu
            rstd = lax.rsqrt(var + 1e-5)
            g = jnp.sum(dyv * ((xv - mu) * rstd), axis=0, keepdims=True)
            b = jnp.sum(dyv, axis=0, keepdims=True)
            if c == 0:
                acc_ref[0:1, :] = g
                acc_ref[1:2, :] = b
            else:
                acc_ref[0:1, :] += g
                acc_ref[1:2, :] += b

        pl.semaphore_wait(barrier_sem, 3)

        rdmas = []
        for k, p in enumerate(peers):
            r = pltpu.make_async_remote_copy(
                src_ref=acc_ref,
                dst_ref=recv_ref.at[k],
                send_sem=send_sems.at[k],
                recv_sem=recv_sems.at[k],
                device_id=p,
                device_id_type=pl.DeviceIdType.MESH,
            )
            r.start()
            rdmas.append(r)
        for r in rdmas:
            r.wait()

        out_ref[:, :] = (
            (acc_ref[:, :] + recv_ref[0]) + (recv_ref[1] + recv_ref[2])
        )

    return pl.pallas_call(
        body,
        out_shape=jax.ShapeDtypeStruct((2, d), jnp.float32),
        in_specs=[
            pl.BlockSpec(memory_space=pltpu.MemorySpace.HBM),
            pl.BlockSpec(memory_space=pltpu.MemorySpace.HBM),
        ],
        out_specs=pl.BlockSpec(memory_space=pltpu.VMEM),
        scratch_shapes=[
            pltpu.VMEM((NCHUNK, cm, d), jnp.float32),
            pltpu.VMEM((NCHUNK, cm, d), jnp.float32),
            pltpu.VMEM((2, d), jnp.float32),
            pltpu.VMEM((3, 2, d), jnp.float32),
            pltpu.SemaphoreType.DMA((2, NCHUNK)),
            pltpu.SemaphoreType.DMA((3,)),
            pltpu.SemaphoreType.DMA((3,)),
        ],
        compiler_params=pltpu.CompilerParams(
            collective_id=0, vmem_limit_bytes=60 * 1024 * 1024
        ),
    )(x, dy)


# device time: 13548 ns/iter; 1.2610x vs baseline; 1.0242x over previous
import jax
import jax.numpy as jnp
from jax import lax
from jax.experimental import pallas as pl
from jax.experimental.pallas import tpu as pltpu

NCHUNK = 8


def kernel(x, dy, gamma):
    m, d = x.shape
    half = m // 2
    cm = half // NCHUNK

    def body(x_hbm, dy_hbm, out_ref, xbuf, dybuf, acc_ref, recv_ref,
             copy_sems, send_sems, recv_sems):
        my_x = lax.axis_index("x")
        my_y = lax.axis_index("y")
        base = my_y * half
        peers = [(1 - my_x, my_y), (my_x, 1 - my_y), (1 - my_x, 1 - my_y)]

        copies = []
        for c in range(NCHUNK):
            cx = pltpu.make_async_copy(
                x_hbm.at[pl.ds(base + c * cm, cm), :], xbuf.at[c],
                copy_sems.at[0, c],
            )
            cd = pltpu.make_async_copy(
                dy_hbm.at[pl.ds(base + c * cm, cm), :], dybuf.at[c],
                copy_sems.at[1, c],
            )
            cx.start()
            cd.start()
            copies.append((cx, cd))

        barrier_sem = pltpu.get_barrier_semaphore()
        for p in peers:
            pl.semaphore_signal(
                barrier_sem, inc=1, device_id=p,
                device_id_type=pl.DeviceIdType.MESH,
            )

        for c in range(NCHUNK):
            cx, cd = copies[c]
            cx.wait()
            cd.wait()
            xv = xbuf[c]
            dyv = dybuf[c]
            mu = jnp.mean(xv, axis=1, keepdims=True)
            var = jnp.mean(xv * xv, axis=1, keepdims=True) - mu * mu
            rstd = lax.rsqrt(var + 1e-5)
            g = jnp.sum(dyv * ((xv - mu) * rstd), axis=0, keepdims=True)
            b = jnp.sum(dyv, axis=0, keepdims=True)
            if c == 0:
                acc_ref[0:1, :] = g
                acc_ref[1:2, :] = b
            else:
                acc_ref[0:1, :] += g
                acc_ref[1:2, :] += b

        pl.semaphore_wait(barrier_sem, 3)

        rdmas = []
        for k, p in enumerate(peers):
            r = pltpu.make_async_remote_copy(
                src_ref=acc_ref,
                dst_ref=recv_ref.at[k],
                send_sem=send_sems.at[k],
                recv_sem=recv_sems.at[k],
                device_id=p,
                device_id_type=pl.DeviceIdType.MESH,
            )
            r.start()
            rdmas.append(r)
        for r in rdmas:
            r.wait()

        out_ref[:, :] = (
            (acc_ref[:, :] + recv_ref[0]) + (recv_ref[1] + recv_ref[2])
        )

    return pl.pallas_call(
        body,
        out_shape=jax.ShapeDtypeStruct((2, d), jnp.float32),
        in_specs=[
            pl.BlockSpec(memory_space=pltpu.MemorySpace.HBM),
            pl.BlockSpec(memory_space=pltpu.MemorySpace.HBM),
        ],
        out_specs=pl.BlockSpec(memory_space=pltpu.VMEM),
        scratch_shapes=[
            pltpu.VMEM((NCHUNK, cm, d), jnp.float32),
            pltpu.VMEM((NCHUNK, cm, d), jnp.float32),
            pltpu.VMEM((2, d), jnp.float32),
            pltpu.VMEM((3, 2, d), jnp.float32),
            pltpu.SemaphoreType.DMA((2, NCHUNK)),
            pltpu.SemaphoreType.DMA((3,)),
            pltpu.SemaphoreType.DMA((3,)),
        ],
        compiler_params=pltpu.CompilerParams(
            collective_id=0, vmem_limit_bytes=60 * 1024 * 1024
        ),
    )(x, dy)
